# baseline (speedup 1.0000x reference)
import jax
import jax.numpy as jnp
from jax import lax
from jax.experimental import pallas as pl
from jax.experimental.pallas import tpu as pltpu
from jax.experimental.pallas import tpu_sc as plsc

B, L, R = 4096, 7, 10
CDIM = 100
DW = 128
NLR = L + R
CPAD = 112
NW = 32
QW = B // NW
NQ = 8
NCHUNK = B // NQ
CH_W = QW // NQ
NROW = NLR * NQ
NLPR = 7
NLIN = NROW * NLPR
NGC = 8
NOFF = 144
U_LINES = 65536 * CDIM // 16
QLIN = QW * NLPR
NGQ = 7


def _rsqrt_vec(n):
    n = jnp.maximum(n, 1e-24)
    i = lax.bitcast_convert_type(n, jnp.int32)
    i = jnp.int32(0x5F3759DF) - (i >> 1)
    y = lax.bitcast_convert_type(i, jnp.float32)
    for _ in range(3):
        y = y * (1.5 - 0.5 * n * y * y)
    return y


def _gather_q_body(lines_hbm, offs_hbm, tl_hbm, out_hbm,
                   lidx_v, off_v, buf_v, rows_v, sem):
    wid = lax.axis_index("s") * 2 + lax.axis_index("c")
    iota = lax.iota(jnp.int32, 16)
    pltpu.sync_copy(lines_hbm.at[wid], lidx_v)
    pltpu.sync_copy(offs_hbm.at[wid], off_v)
    for g in range(NGQ):
        pltpu.async_copy(tl_hbm.at[lidx_v.at[g]],
                         buf_v.at[pl.ds(g * 128, 128)], sem)
    for g in range(NGQ):
        pltpu.make_async_copy(tl_hbm.at[lidx_v.at[g]],
                              buf_v.at[pl.ds(g * 128, 128)], sem).wait()

    def rb(m, _):
        ochunk = off_v[pl.ds((m // 16) * 16, 16)]
        off = jnp.sum(jnp.where(iota == (m % 16), ochunk, 0))
        pv = m * NLPR * 16 + off + iota
        colv = pv & 15
        rowb = pv >> 4
        for k in range(7):
            rows_v[m, pl.ds(k * 16, 16)] = plsc.load_gather(
                buf_v, [rowb + k, colv])
        return 0

    lax.fori_loop(0, QW, rb, 0)
    pltpu.sync_copy(rows_v, out_hbm.at[pl.ds(wid * QW, QW)])


_gather_q = pl.kernel(
    _gather_q_body,
    mesh=plsc.VectorSubcoreMesh(core_axis_name="c", subcore_axis_name="s"),
    compiler_params=pltpu.CompilerParams(use_tc_tiling_on_sc=False, needs_layout_passes=False),
    out_type=jax.ShapeDtypeStruct((B, CPAD), jnp.float32),
    scratch_types=[
        pltpu.VMEM((NGQ, 128), jnp.int32),
        pltpu.VMEM((QW,), jnp.int32),
        pltpu.VMEM((QLIN, 16), jnp.float32),
        pltpu.VMEM((QW, CPAD), jnp.float32),
        pltpu.SemaphoreType.DMA,
    ],
)


def _proj_body(q_ref, lw_ref, rw_ref, ul_ref, ur_ref):
    q = q_ref[...]
    cols = lax.broadcasted_iota(jnp.int32, q.shape, 1)
    q = jnp.where(cols < CDIM, q, 0.0)
    s = jnp.sum(q * q, axis=1, keepdims=True)
    qn = q / jnp.maximum(jnp.sqrt(s), 1e-12)
    ul_ref[...] = jnp.dot(qn, lw_ref[...], preferred_element_type=jnp.float32)
    ur_ref[...] = jnp.dot(qn, rw_ref[...], preferred_element_type=jnp.float32)


_proj = pl.pallas_call(
    _proj_body,
    grid=(8,),
    in_specs=[
        pl.BlockSpec((B // 8, CPAD), lambda i: (i, 0)),
        pl.BlockSpec((CPAD, CPAD), lambda i: (0, 0)),
        pl.BlockSpec((CPAD, CPAD), lambda i: (0, 0)),
    ],
    out_specs=[
        pl.BlockSpec((B // 8, CPAD), lambda i: (i, 0)),
        pl.BlockSpec((B // 8, CPAD), lambda i: (i, 0)),
    ],
    out_shape=[
        jax.ShapeDtypeStruct((B, CPAD), jnp.float32),
        jax.ShapeDtypeStruct((B, CPAD), jnp.float32),
    ],
)


def _main_body(tl_hbm, wvec_hbm, clines_hbm, coffs_hbm, widx_hbm,
               ul_hbm, ur_hbm, gp_hbm, out_hbm,
               widx_v, gp_v, out_v,
               cl_a, cl_b, co_a, co_b, crows_a, crows_b, wrows_a, wrows_b,
               ul_a, ul_b, ur_a, ur_b,
               sem_ca, sem_cb, sem_wa, sem_wb):
    wid = lax.axis_index("s") * 2 + lax.axis_index("c")
    c0 = wid * CH_W
    pltpu.sync_copy(gp_hbm, gp_v)
    pltpu.sync_copy(widx_hbm.at[pl.ds(c0, CH_W)], widx_v)

    iota = lax.iota(jnp.int32, 16)
    lmask = iota < L
    rmask = iota < R
    nmask6 = jnp.where(iota < 4, 1.0, 0.0)

    g0l = gp_v[0, :]
    g0r = gp_v[1, :]
    g1l = gp_v[2, :]
    g1r = gp_v[3, :]
    gbv = gp_v[4, :]
    gb0 = gbv[0]
    gb1 = gbv[1]

    def fire(cl, cl_v, co_v, crows, wrows, ul_v, ur_v, sem_c, sem_w):
        cl = jnp.minimum(cl, CH_W - 1)
        c = c0 + cl
        pltpu.sync_copy(clines_hbm.at[c], cl_v)
        pltpu.sync_copy(coffs_hbm.at[c], co_v)
        pltpu.sync_copy(ul_hbm.at[pl.ds(c * NQ, NQ)], ul_v)
        pltpu.sync_copy(ur_hbm.at[pl.ds(c * NQ, NQ)], ur_v)
        for g in range(NGC):
            pltpu.async_copy(tl_hbm.at[cl_v.at[g]],
                             crows.at[pl.ds(g * 128, 128)], sem_c)
        for j in range(2):
            pltpu.async_copy(wvec_hbm.at[widx_v.at[cl, j]],
                             wrows.at[pl.ds(j * 68, 68)], sem_w)

    def drain(cl, cl_v, crows, wrows, sem_c, sem_w):
        cl = jnp.minimum(cl, CH_W - 1)
        for g in range(NGC):
            pltpu.make_async_copy(tl_hbm.at[cl_v.at[g]],
                                  crows.at[pl.ds(g * 128, 128)], sem_c).wait()
        for j in range(2):
            pltpu.make_async_copy(wvec_hbm.at[widx_v.at[cl, j]],
                                  wrows.at[pl.ds(j * 68, 68)], sem_w).wait()

    def compute(cl, co_v, crows, wrows, ul_v, ur_v):
        def q_body(q, _):
            lq = cl * NQ + q

            def srow(i, roff, carry):
                sv, nv = carry
                r = (roff + i) * NQ + q
                ochunk = co_v[pl.ds((r // 16) * 16, 16)]
                off = jnp.sum(jnp.where(iota == (r % 16), ochunk, 0))
                pv = r * (NLPR * 16) + off + iota
                colv = pv & 15
                rowb = pv >> 4
                u_v = ul_v if roff == 0 else ur_v
                acc_s = jnp.zeros((16,), jnp.float32)
                acc_n = jnp.zeros((16,), jnp.float32)
                for k in range(7):
                    rv = plsc.load_gather(crows, [rowb + k, colv])
                    uv = u_v[q, pl.ds(k * 16, 16)]
                    acc_s = acc_s + rv * uv
                    if k < 6:
                        acc_n = acc_n + rv * rv
                    else:
                        rvm = rv * nmask6
                        acc_n = acc_n + rvm * rvm
                sv = jnp.where(iota == i, jnp.full((16,), jnp.sum(acc_s)), sv)
                nv = jnp.where(iota == i, jnp.full((16,), jnp.sum(acc_n)), nv)
                return sv, nv

            zeros = jnp.zeros((16,), jnp.float32)
            sl, nl = lax.fori_loop(
                0, L, lambda i, cy: srow(i, 0, cy), (zeros, zeros))
            sr, nr = lax.fori_loop(
                0, R, lambda i, cy: srow(i, L, cy), (zeros, zeros))

            zl = jnp.where(lmask, sl * _rsqrt_vec(nl), -1e9)
            zr = jnp.where(rmask, sr * _rsqrt_vec(nr), -1e9)
            el = jnp.exp(zl - jnp.max(zl))
            er = jnp.exp(zr - jnp.max(zr))
            la = el / jnp.full((16,), jnp.sum(el))
            ra = er / jnp.full((16,), jnp.sum(er))
            gl0 = jnp.sum(la * g0l) + jnp.sum(ra * g0r) + gb0
            gl1 = jnp.sum(la * g1l) + jnp.sum(ra * g1r) + gb1
            ev = jnp.exp(jnp.full((16,), gl1 - gl0, jnp.float32))
            a0 = 1.0 / (1.0 + ev)
            la_s = la * a0
            ra_s = ra * (1.0 - a0)

            def pool(i, accs, wvv, roff):
                w = jnp.sum(jnp.where(iota == i, wvv, 0.0))
                r = (roff + i) * NQ + q
                return tuple(accs[k] + w * wrows[r, pl.ds(k * 16, 16)]
                             for k in range(8))

            accs0 = tuple(jnp.zeros((16,), jnp.float32) for _ in range(8))
            accs = lax.fori_loop(0, L, lambda i, a: pool(i, a, la_s, 0), accs0)
            accs = lax.fori_loop(0, R, lambda i, a: pool(i, a, ra_s, L), accs)
            for k in range(8):
                out_v[lq, pl.ds(k * 16, 16)] = accs[k]
            return 0

        lax.fori_loop(0, NQ, q_body, 0)

    fire(0, cl_a, co_a, crows_a, wrows_a, ul_a, ur_a, sem_ca, sem_wa)

    def step(s, _):
        ca = 2 * s
        cb = 2 * s + 1
        fire(cb, cl_b, co_b, crows_b, wrows_b, ul_b, ur_b, sem_cb, sem_wb)
        drain(ca, cl_a, crows_a, wrows_a, sem_ca, sem_wa)
        compute(ca, co_a, crows_a, wrows_a, ul_a, ur_a)
        fire(ca + 2, cl_a, co_a, crows_a, wrows_a, ul_a, ur_a, sem_ca, sem_wa)
        drain(cb, cl_b, crows_b, wrows_b, sem_cb, sem_wb)
        compute(cb, co_b, crows_b, wrows_b, ul_b, ur_b)
        return 0

    lax.fori_loop(0, CH_W // 2, step, 0)
    drain(CH_W - 1, cl_a, crows_a, wrows_a, sem_ca, sem_wa)

    pltpu.sync_copy(out_v, out_hbm.at[pl.ds(wid * QW, QW)])


_char_main = pl.kernel(
    _main_body,
    mesh=plsc.VectorSubcoreMesh(core_axis_name="c", subcore_axis_name="s"),
    compiler_params=pltpu.CompilerParams(use_tc_tiling_on_sc=False, needs_layout_passes=False),
    out_type=jax.ShapeDtypeStruct((B, DW), jnp.float32),
    scratch_types=[
        pltpu.VMEM((CH_W, 2, 68), jnp.int32),
        pltpu.VMEM((8, 16), jnp.float32),
        pltpu.VMEM((QW, DW), jnp.float32),
        pltpu.VMEM((NGC, 128), jnp.int32),
        pltpu.VMEM((NGC, 128), jnp.int32),
        pltpu.VMEM((NOFF,), jnp.int32),
        pltpu.VMEM((NOFF,), jnp.int32),
        pltpu.VMEM((NGC * 128, 16), jnp.float32),
        pltpu.VMEM((NGC * 128, 16), jnp.float32),
        pltpu.VMEM((NROW, DW), jnp.float32),
        pltpu.VMEM((NROW, DW), jnp.float32),
        pltpu.VMEM((NQ, CPAD), jnp.float32),
        pltpu.VMEM((NQ, CPAD), jnp.float32),
        pltpu.VMEM((NQ, CPAD), jnp.float32),
        pltpu.VMEM((NQ, CPAD), jnp.float32),
        pltpu.SemaphoreType.DMA,
        pltpu.SemaphoreType.DMA,
        pltpu.SemaphoreType.DMA,
        pltpu.SemaphoreType.DMA,
    ],
)


def _line_indices(t):
    w0 = t * CDIM
    l0 = w0 // 16
    off = w0 % 16
    lines = l0[..., None] + jnp.arange(NLPR, dtype=jnp.int32)
    return lines, off


def kernel(ce_raw, qidx, lixs_char, rixs_char, lixs_w, rixs_w, wvec, lW, rW, gW, gb):
    qidx = qidx.astype(jnp.int32)
    cidx = jnp.concatenate([lixs_char, rixs_char], axis=1).astype(jnp.int32)
    widx = jnp.concatenate([lixs_w, rixs_w], axis=1).astype(jnp.int32)
    tl = ce_raw.reshape(U_LINES, 16)

    cidx_ch = cidx.reshape(NCHUNK, NQ, NLR).transpose(0, 2, 1).reshape(NCHUNK, NROW)
    widx_ch = widx.reshape(NCHUNK, NQ, NLR).transpose(0, 2, 1).reshape(NCHUNK, 2, 68)
    clines, coffs = _line_indices(cidx_ch)
    clines = jnp.pad(clines.reshape(NCHUNK, NLIN),
                     ((0, 0), (0, NGC * 128 - NLIN))).reshape(NCHUNK, NGC, 128)
    coffs = jnp.pad(coffs, ((0, 0), (0, NOFF - NROW)))

    qlines, qoffs = _line_indices(qidx)
    qlines = qlines.reshape(NW, NGQ * 128).reshape(NW, NGQ, 128)
    qoffs = qoffs.reshape(NW, QW)

    lWp = jnp.pad(lW.T, ((0, CPAD - CDIM), (0, CPAD - CDIM)))
    rWp = jnp.pad(rW.T, ((0, CPAD - CDIM), (0, CPAD - CDIM)))
    gp = jnp.zeros((8, 16), jnp.float32)
    gp = gp.at[0, :L].set(gW[:L, 0]).at[1, :R].set(gW[L:, 0])
    gp = gp.at[2, :L].set(gW[:L, 1]).at[3, :R].set(gW[L:, 1])
    gp = gp.at[4, 0].set(gb[0]).at[4, 1].set(gb[1])

    qraw = _gather_q(qlines, qoffs, tl)
    ul, ur = _proj(qraw, lWp, rWp)
    return _char_main(tl, wvec, clines, coffs, widx_ch, ul, ur, gp)


# speedup vs baseline: 1.8939x; 1.8939x over previous
import jax
import jax.numpy as jnp
from jax import lax
from jax.experimental import pallas as pl
from jax.experimental.pallas import tpu as pltpu
from jax.experimental.pallas import tpu_sc as plsc

B, L, R = 4096, 7, 10
CDIM = 100
DW = 128
NLR = L + R
CPAD = 112
NW = 32
QW = B // NW
NQ = 8
NCHUNK = B // NQ
CH_W = QW // NQ
NROW = NLR * NQ
NSEG = 2
SEG = NROW // NSEG


def _rsqrt_vec(n):
    n = jnp.maximum(n, 1e-24)
    i = lax.bitcast_convert_type(n, jnp.int32)
    i = jnp.int32(0x5F3759DF) - (i >> 1)
    y = lax.bitcast_convert_type(i, jnp.float32)
    for _ in range(3):
        y = y * (1.5 - 0.5 * n * y * y)
    return y


def _gather_q_body(idx_hbm, tab_hbm, out_hbm, idx_v, rows_v, sem):
    wid = lax.axis_index("s") * 2 + lax.axis_index("c")
    base = wid * QW
    pltpu.sync_copy(idx_hbm.at[pl.ds(base, QW)], idx_v)
    pltpu.async_copy(tab_hbm.at[idx_v], rows_v, sem).wait()
    pltpu.sync_copy(rows_v, out_hbm.at[pl.ds(base, QW)])


_gather_q = pl.kernel(
    _gather_q_body,
    mesh=plsc.VectorSubcoreMesh(core_axis_name="c", subcore_axis_name="s"),
    compiler_params=pltpu.CompilerParams(use_tc_tiling_on_sc=False, needs_layout_passes=False),
    out_type=jax.ShapeDtypeStruct((B, CPAD), jnp.float32),
    scratch_types=[
        pltpu.VMEM((QW,), jnp.int32),
        pltpu.VMEM((QW, CPAD), jnp.float32),
        pltpu.SemaphoreType.DMA,
    ],
)


def _proj_body(q_ref, lw_ref, rw_ref, ul_ref, ur_ref):
    q = q_ref[...]
    s = jnp.sum(q * q, axis=1, keepdims=True)
    qn = q / jnp.maximum(jnp.sqrt(s), 1e-12)
    ul_ref[...] = jnp.dot(qn, lw_ref[...], preferred_element_type=jnp.float32)
    ur_ref[...] = jnp.dot(qn, rw_ref[...], preferred_element_type=jnp.float32)


_proj = pl.pallas_call(
    _proj_body,
    grid=(8,),
    in_specs=[
        pl.BlockSpec((B // 8, CPAD), lambda i: (i, 0)),
        pl.BlockSpec((CPAD, CPAD), lambda i: (0, 0)),
        pl.BlockSpec((CPAD, CPAD), lambda i: (0, 0)),
    ],
    out_specs=[
        pl.BlockSpec((B // 8, CPAD), lambda i: (i, 0)),
        pl.BlockSpec((B // 8, CPAD), lambda i: (i, 0)),
    ],
    out_shape=[
        jax.ShapeDtypeStruct((B, CPAD), jnp.float32),
        jax.ShapeDtypeStruct((B, CPAD), jnp.float32),
    ],
)


def _main_body(ce_hbm, wvec_hbm, cidx_hbm, widx_hbm, ul_hbm, ur_hbm, gp_hbm,
               out_hbm,
               cidx_v, widx_v, crows_a, crows_b, wrows_a, wrows_b,
               ul_v, ur_v, gp_v, out_v,
               sem_ca, sem_cb, sem_wa, sem_wb):
    wid = lax.axis_index("s") * 2 + lax.axis_index("c")
    c0 = wid * CH_W
    pltpu.sync_copy(gp_hbm, gp_v)
    pltpu.sync_copy(cidx_hbm.at[pl.ds(c0, CH_W)], cidx_v)
    pltpu.sync_copy(widx_hbm.at[pl.ds(c0, CH_W)], widx_v)
    pltpu.sync_copy(ul_hbm.at[pl.ds(wid * QW, QW)], ul_v)
    pltpu.sync_copy(ur_hbm.at[pl.ds(wid * QW, QW)], ur_v)

    iota = lax.iota(jnp.int32, 16)
    lmask = iota < L
    rmask = iota < R

    g0l = gp_v[0, :]
    g0r = gp_v[1, :]
    g1l = gp_v[2, :]
    g1r = gp_v[3, :]
    gbv = gp_v[4, :]
    gb0 = gbv[0]
    gb1 = gbv[1]

    def fire(cl, crows, wrows, sem_c, sem_w):
        cl = jnp.minimum(cl, CH_W - 1)
        for j in range(NSEG):
            pltpu.async_copy(ce_hbm.at[cidx_v.at[cl, j]],
                             crows.at[pl.ds(j * SEG, SEG)], sem_c)
        for j in range(NSEG):
            pltpu.async_copy(wvec_hbm.at[widx_v.at[cl, j]],
                             wrows.at[pl.ds(j * SEG, SEG)], sem_w)

    def drain(cl, crows, wrows, sem_c, sem_w):
        cl = jnp.minimum(cl, CH_W - 1)
        for j in range(NSEG):
            pltpu.make_async_copy(ce_hbm.at[cidx_v.at[cl, j]],
                                  crows.at[pl.ds(j * SEG, SEG)], sem_c).wait()
        for j in range(NSEG):
            pltpu.make_async_copy(wvec_hbm.at[widx_v.at[cl, j]],
                                  wrows.at[pl.ds(j * SEG, SEG)], sem_w).wait()

    def compute(cl, crows, wrows):
        def q_body(q, _):
            lq = cl * NQ + q
            ulc = [ul_v[lq, pl.ds(k * 16, 16)] for k in range(7)]
            urc = [ur_v[lq, pl.ds(k * 16, 16)] for k in range(7)]

            def srow(i, roff, uc, carry):
                sv, nv = carry
                r = (roff + i) * NQ + q
                acc_s = jnp.zeros((16,), jnp.float32)
                acc_n = jnp.zeros((16,), jnp.float32)
                for k in range(7):
                    rv = crows[r, pl.ds(k * 16, 16)]
                    acc_s = acc_s + rv * uc[k]
                    acc_n = acc_n + rv * rv
                sv = jnp.where(iota == i, jnp.full((16,), jnp.sum(acc_s)), sv)
                nv = jnp.where(iota == i, jnp.full((16,), jnp.sum(acc_n)), nv)
                return sv, nv

            zeros = jnp.zeros((16,), jnp.float32)
            sl, nl = lax.fori_loop(
                0, L, lambda i, cy: srow(i, 0, ulc, cy), (zeros, zeros))
            sr, nr = lax.fori_loop(
                0, R, lambda i, cy: srow(i, L, urc, cy), (zeros, zeros))

            zl = jnp.where(lmask, sl * _rsqrt_vec(nl), -1e9)
            zr = jnp.where(rmask, sr * _rsqrt_vec(nr), -1e9)
            el = jnp.exp(zl - jnp.max(zl))
            er = jnp.exp(zr - jnp.max(zr))
            la = el / jnp.full((16,), jnp.sum(el))
            ra = er / jnp.full((16,), jnp.sum(er))
            gl0 = jnp.sum(la * g0l) + jnp.sum(ra * g0r) + gb0
            gl1 = jnp.sum(la * g1l) + jnp.sum(ra * g1r) + gb1
            ev = jnp.exp(jnp.full((16,), gl1 - gl0, jnp.float32))
            a0 = 1.0 / (1.0 + ev)
            la_s = la * a0
            ra_s = ra * (1.0 - a0)

            def pool(i, accs, wvv, roff):
                w = jnp.sum(jnp.where(iota == i, wvv, 0.0))
                r = (roff + i) * NQ + q
                return tuple(accs[k] + w * wrows[r, pl.ds(k * 16, 16)]
                             for k in range(8))

            accs0 = tuple(jnp.zeros((16,), jnp.float32) for _ in range(8))
            accs = lax.fori_loop(0, L, lambda i, a: pool(i, a, la_s, 0), accs0)
            accs = lax.fori_loop(0, R, lambda i, a: pool(i, a, ra_s, L), accs)
            for k in range(8):
                out_v[lq, pl.ds(k * 16, 16)] = accs[k]
            return 0

        lax.fori_loop(0, NQ, q_body, 0)

    fire(0, crows_a, wrows_a, sem_ca, sem_wa)

    def step(s, _):
        ca = 2 * s
        cb = 2 * s + 1
        fire(cb, crows_b, wrows_b, sem_cb, sem_wb)
        drain(ca, crows_a, wrows_a, sem_ca, sem_wa)
        compute(ca, crows_a, wrows_a)
        fire(ca + 2, crows_a, wrows_a, sem_ca, sem_wa)
        drain(cb, crows_b, wrows_b, sem_cb, sem_wb)
        compute(cb, crows_b, wrows_b)
        return 0

    lax.fori_loop(0, CH_W // 2, step, 0)
    drain(CH_W - 1, crows_a, wrows_a, sem_ca, sem_wa)

    pltpu.sync_copy(out_v, out_hbm.at[pl.ds(wid * QW, QW)])


_char_main = pl.kernel(
    _main_body,
    mesh=plsc.VectorSubcoreMesh(core_axis_name="c", subcore_axis_name="s"),
    compiler_params=pltpu.CompilerParams(use_tc_tiling_on_sc=False, needs_layout_passes=False),
    out_type=jax.ShapeDtypeStruct((B, DW), jnp.float32),
    scratch_types=[
        pltpu.VMEM((CH_W, NSEG, SEG), jnp.int32),
        pltpu.VMEM((CH_W, NSEG, SEG), jnp.int32),
        pltpu.VMEM((NROW, CPAD), jnp.float32),
        pltpu.VMEM((NROW, CPAD), jnp.float32),
        pltpu.VMEM((NROW, DW), jnp.float32),
        pltpu.VMEM((NROW, DW), jnp.float32),
        pltpu.VMEM((QW, CPAD), jnp.float32),
        pltpu.VMEM((QW, CPAD), jnp.float32),
        pltpu.VMEM((8, 16), jnp.float32),
        pltpu.VMEM((QW, DW), jnp.float32),
        pltpu.SemaphoreType.DMA,
        pltpu.SemaphoreType.DMA,
        pltpu.SemaphoreType.DMA,
        pltpu.SemaphoreType.DMA,
    ],
)


def kernel(ce_raw, qidx, lixs_char, rixs_char, lixs_w, rixs_w, wvec, lW, rW, gW, gb):
    qidx = qidx.astype(jnp.int32)
    cidx = jnp.concatenate([lixs_char, rixs_char], axis=1).astype(jnp.int32)
    widx = jnp.concatenate([lixs_w, rixs_w], axis=1).astype(jnp.int32)
    cidx_ch = cidx.reshape(NCHUNK, NQ, NLR).transpose(0, 2, 1).reshape(NCHUNK, NSEG, SEG)
    widx_ch = widx.reshape(NCHUNK, NQ, NLR).transpose(0, 2, 1).reshape(NCHUNK, NSEG, SEG)
    cep = jnp.pad(ce_raw, ((0, 0), (0, CPAD - CDIM)))
    lWp = jnp.pad(lW.T, ((0, CPAD - CDIM), (0, CPAD - CDIM)))
    rWp = jnp.pad(rW.T, ((0, CPAD - CDIM), (0, CPAD - CDIM)))
    gp = jnp.zeros((8, 16), jnp.float32)
    gp = gp.at[0, :L].set(gW[:L, 0]).at[1, :R].set(gW[L:, 0])
    gp = gp.at[2, :L].set(gW[:L, 1]).at[3, :R].set(gW[L:, 1])
    gp = gp.at[4, 0].set(gb[0]).at[4, 1].set(gb[1])
    qraw = _gather_q(qidx, cep)
    ul, ur = _proj(qraw, lWp, rWp)
    return _char_main(cep, wvec, cidx_ch, widx_ch, ul, ur, gp)
